# Initial kernel scaffold; baseline (speedup 1.0000x reference)
import jax
import jax.numpy as jnp
from jax.experimental import pallas as pl


def kernel(x, adj_indices, adj_values, embedding):
    raise NotImplementedError("write your pallas kernel here")


# speedup vs baseline: 3.0441x; 3.0441x over previous
import functools

import jax
import jax.numpy as jnp
from jax import lax
from jax.experimental import pallas as pl
from jax.experimental.pallas import tpu as pltpu
from jax.experimental.pallas import tpu_sc as plsc

_N = 10000
_D = 128
_NNZ = 320000
_B = 1024
_L = 50
_M = 10240
_LP = 64

_NC = 2
_NS = 16
_NW = _NC * _NS

_CHUNK = 128
_NCHUNK = _NNZ // _CHUNK
_CH_PER_SC = _NCHUNK // _NC
_T_STEPS = (_CH_PER_SC + _NS - 1) // _NS

_PB = _B // _NW

_VMESH = plsc.VectorSubcoreMesh(core_axis_name="c", subcore_axis_name="s")


def _scatter_body(src_hbm, rows_hbm, cols_hbm, vals_hbm, out_hbm,
                  acc, gbuf, rows_v, cols_v, vals_v, sem):
    c = lax.axis_index("c")
    s = lax.axis_index("s")

    def _z(r, _):
        for d in range(8):
            gbuf[r, pl.ds(d * 16, 16)] = jnp.zeros((16,), jnp.float32)
        return 0
    lax.fori_loop(0, _CHUNK, _z, 0)
    rows_per_sub = _M // _NS
    for k in range(rows_per_sub // _CHUNK):
        pltpu.sync_copy(gbuf, acc.at[pl.ds(s * rows_per_sub + k * _CHUNK, _CHUNK)])
    plsc.subcore_barrier()

    def _step(t, _):
        j = s + _NS * t

        @pl.when(j < _CH_PER_SC)
        def _():
            chunk = c * _CH_PER_SC + j
            pltpu.sync_copy(rows_hbm.at[chunk], rows_v.at[0])
            pltpu.sync_copy(cols_hbm.at[chunk], cols_v)
            pltpu.sync_copy(vals_hbm.at[chunk], vals_v)
            for i in range(8):
                cols_v[pl.ds(i * 16, 16)] = cols_v[pl.ds(i * 16, 16)] + 1
                rows_v[0, pl.ds(i * 16, 16)] = rows_v[0, pl.ds(i * 16, 16)] + 1
            pltpu.async_copy(src_hbm.at[cols_v], gbuf, sem).wait()

            def _scale(g, _):
                val16 = vals_v[pl.ds(g * 16, 16)]
                for l in range(16):
                    v16 = jnp.full((16,), val16[l])
                    e = g * 16 + l
                    for d in range(8):
                        gbuf[e, pl.ds(d * 16, 16)] = (
                            gbuf[e, pl.ds(d * 16, 16)] * v16)
                return 0
            lax.fori_loop(0, _CHUNK // 16, _scale, 0)
            pltpu.sync_copy(gbuf, acc.at[rows_v.at[0]], add=True)
        return 0
    lax.fori_loop(0, _T_STEPS, _step, 0)
    plsc.subcore_barrier()

    for k in range(rows_per_sub // _CHUNK):
        r0 = s * rows_per_sub + k * _CHUNK
        pltpu.sync_copy(acc.at[pl.ds(r0, _CHUNK)], gbuf)
        pltpu.sync_copy(gbuf, out_hbm.at[c].at[pl.ds(r0, _CHUNK)])


_scatter_kernel = functools.partial(
    pl.kernel,
    out_type=jax.ShapeDtypeStruct((_NC, _M, _D), jnp.float32),
    mesh=_VMESH,
    scratch_types=[
        pltpu.VMEM_SHARED((_M, _D), jnp.float32),
        pltpu.VMEM((_CHUNK, _D), jnp.float32),
        pltpu.VMEM((1, _CHUNK), jnp.int32),
        pltpu.VMEM((_CHUNK,), jnp.int32),
        pltpu.VMEM((_CHUNK,), jnp.float32),
        pltpu.SemaphoreType.DMA,
    ],
)(_scatter_body)


def _combine_body(a_ref, b_ref, o_ref):
    o_ref[...] = jnp.maximum(a_ref[...] + b_ref[...], 0.0)


def _mean_body(e_ref, e1_ref, a_ref, b_ref, o_ref):
    e2 = jnp.maximum(a_ref[...] + b_ref[...], 0.0)
    o_ref[...] = (e_ref[...] + e1_ref[...] + e2) * jnp.float32(1.0 / 3.0)


_BR = 1280

_combine_kernel = pl.pallas_call(
    _combine_body,
    out_shape=jax.ShapeDtypeStruct((_M, _D), jnp.float32),
    grid=(_M // _BR,),
    in_specs=[pl.BlockSpec((_BR, _D), lambda i: (i, 0))] * 2,
    out_specs=pl.BlockSpec((_BR, _D), lambda i: (i, 0)),
)

_mean_kernel = pl.pallas_call(
    _mean_body,
    out_shape=jax.ShapeDtypeStruct((_M, _D), jnp.float32),
    grid=(_M // _BR,),
    in_specs=[pl.BlockSpec((_BR, _D), lambda i: (i, 0))] * 4,
    out_specs=pl.BlockSpec((_BR, _D), lambda i: (i, 0)),
)


def _inv_cnt_body(x_ref, o_ref):
    valid = (x_ref[...] != 0).astype(jnp.float32)
    cnt = jnp.maximum(jnp.sum(valid, axis=1, keepdims=True), 1.0)
    o_ref[...] = jnp.broadcast_to(1.0 / cnt, (_B, 16))


_inv_cnt_kernel = pl.pallas_call(
    _inv_cnt_body,
    out_shape=jax.ShapeDtypeStruct((_B, 16), jnp.float32),
)


def _gather_mean_body(tab_hbm, x_hbm, inv_hbm, out_hbm,
                      xb_v, inv_v, gbuf, obuf, sem):
    c = lax.axis_index("c")
    s = lax.axis_index("s")
    w = s * _NC + c
    base = w * _PB
    pltpu.sync_copy(x_hbm.at[pl.ds(base, _PB)], xb_v)
    pltpu.sync_copy(inv_hbm.at[pl.ds(base, _PB)], inv_v)

    zeros = jnp.zeros((16,), jnp.float32)

    def _patient(b, _):
        pltpu.async_copy(tab_hbm.at[xb_v.at[b]], gbuf, sem).wait()
        inv = inv_v[b]

        def _acc(e, accs):
            return tuple(a + gbuf[e, pl.ds(d * 16, 16)]
                         for d, a in enumerate(accs))
        accs = lax.fori_loop(0, _LP, _acc, tuple(zeros for _ in range(8)))
        for d in range(8):
            obuf[b, pl.ds(d * 16, 16)] = accs[d] * inv
        return 0
    lax.fori_loop(0, _PB, _patient, 0)
    pltpu.sync_copy(obuf, out_hbm.at[pl.ds(base, _PB)])


_gather_mean_kernel = functools.partial(
    pl.kernel,
    out_type=jax.ShapeDtypeStruct((_B, _D), jnp.float32),
    mesh=_VMESH,
    scratch_types=[
        pltpu.VMEM((_PB, _LP), jnp.int32),
        pltpu.VMEM((_PB, 16), jnp.float32),
        pltpu.VMEM((_LP, _D), jnp.float32),
        pltpu.VMEM((_PB, _D), jnp.float32),
        pltpu.SemaphoreType.DMA,
    ],
)(_gather_mean_body)


@jax.jit
def kernel(x, adj_indices, adj_values, embedding):
    rows2 = adj_indices[0].reshape(_NCHUNK, _CHUNK)
    cols2 = adj_indices[1].reshape(_NCHUNK, _CHUNK)
    vals2 = adj_values.reshape(_NCHUNK, _CHUNK)

    e0 = jnp.zeros((_M, _D), jnp.float32).at[1:_N + 1].set(embedding[1:])
    x_pad = jnp.zeros((_B, _LP), jnp.int32).at[:, :_L].set(x)

    p1 = _scatter_kernel(e0, rows2, cols2, vals2)
    e1 = _combine_kernel(p1[0], p1[1])
    p2 = _scatter_kernel(e1, rows2, cols2, vals2)
    eavg = _mean_kernel(e0, e1, p2[0], p2[1])
    invb = _inv_cnt_kernel(x_pad)
    return _gather_mean_kernel(eavg, x_pad, invb)
